# Initial kernel scaffold; baseline (speedup 1.0000x reference)
import jax
import jax.numpy as jnp
from jax.experimental import pallas as pl


def kernel(h, edge_index, W, b):
    raise NotImplementedError("write your pallas kernel here")


# speedup vs baseline: 29.6255x; 29.6255x over previous
import functools

import jax
import jax.numpy as jnp
from jax import lax
from jax.experimental import pallas as pl
from jax.experimental.pallas import tpu as pltpu
from jax.experimental.pallas import tpu_sc as plsc

N_NODES = 10000
N_EDGES = 320000
D_FEAT = 128

SC_CORES = 2
SC_SUBCORES = 16
SC_LANES = 16
N_WORKERS = SC_CORES * SC_SUBCORES
EDGES_PER_WORKER = N_EDGES // N_WORKERS


def _node_table_body(h_ref, w2_ref, b_ref, out_ref):
    res = jnp.dot(h_ref[...], w2_ref[...], preferred_element_type=jnp.float32)
    col = lax.broadcasted_iota(jnp.int32, res.shape, 1)
    out_ref[...] = res + jnp.where(col == 0, b_ref[0], 0.0)


def _node_table(h, w2, b):
    return pl.pallas_call(
        _node_table_body,
        out_shape=jax.ShapeDtypeStruct((N_NODES, 2), jnp.float32),
        in_specs=[
            pl.BlockSpec(memory_space=pltpu.VMEM),
            pl.BlockSpec(memory_space=pltpu.VMEM),
            pl.BlockSpec(memory_space=pltpu.SMEM),
        ],
        out_specs=pl.BlockSpec(memory_space=pltpu.VMEM),
    )(h, w2, b)


def _edge_scores(pq_flat, src, dst):
    mesh = plsc.VectorSubcoreMesh(core_axis_name="c", subcore_axis_name="s")

    @functools.partial(
        pl.kernel,
        mesh=mesh,
        out_type=jax.ShapeDtypeStruct((N_EDGES,), jnp.float32),
        compiler_params=pltpu.CompilerParams(needs_layout_passes=False),
        scratch_types=[
            pltpu.VMEM((2 * N_NODES,), jnp.float32),
            pltpu.VMEM((EDGES_PER_WORKER,), jnp.int32),
            pltpu.VMEM((EDGES_PER_WORKER,), jnp.int32),
            pltpu.VMEM((EDGES_PER_WORKER,), jnp.float32),
        ],
    )
    def sc_kernel(pq_hbm, src_hbm, dst_hbm, out_hbm, pq_v, src_v, dst_v, o_v):
        wid = lax.axis_index("s") * SC_CORES + lax.axis_index("c")
        base = wid * EDGES_PER_WORKER
        pltpu.sync_copy(pq_hbm, pq_v)
        pltpu.sync_copy(src_hbm.at[pl.ds(base, EDGES_PER_WORKER)], src_v)
        pltpu.sync_copy(dst_hbm.at[pl.ds(base, EDGES_PER_WORKER)], dst_v)

        @pl.loop(0, EDGES_PER_WORKER, step=SC_LANES)
        def _(i):
            sl = pl.ds(i, SC_LANES)
            s2 = src_v[sl] * 2
            d2 = dst_v[sl] * 2 + 1
            o_v[sl] = plsc.load_gather(pq_v, [s2]) + plsc.load_gather(pq_v, [d2])

        pltpu.sync_copy(o_v, out_hbm.at[pl.ds(base, EDGES_PER_WORKER)])

    return sc_kernel(pq_flat, src, dst)


def kernel(h, edge_index, W, b):
    w2 = W.reshape(2, D_FEAT).T
    pq = _node_table(h, w2, b)
    src = edge_index[0].astype(jnp.int32)
    dst = edge_index[1].astype(jnp.int32)
    scores = _edge_scores(pq.reshape(-1), src, dst)
    return scores.reshape(N_EDGES, 1)


# speedup vs baseline: 35.8423x; 1.2098x over previous
import functools

import jax
import jax.numpy as jnp
from jax import lax
from jax.experimental import pallas as pl
from jax.experimental.pallas import tpu as pltpu
from jax.experimental.pallas import tpu_sc as plsc

N_NODES = 10000
N_EDGES = 320000
D_FEAT = 128

SC_CORES = 2
SC_SUBCORES = 16
SC_LANES = 16
N_WORKERS = SC_CORES * SC_SUBCORES
EDGES_PER_WORKER = N_EDGES // N_WORKERS


def _node_table_body(h_ref, w2_ref, b_ref, out_ref):
    res = jnp.dot(h_ref[...], w2_ref[...], preferred_element_type=jnp.float32)
    col = lax.broadcasted_iota(jnp.int32, res.shape, 1)
    out_ref[...] = res + jnp.where(col == 0, b_ref[0], 0.0)


def _node_table(h, w2, b):
    return pl.pallas_call(
        _node_table_body,
        out_shape=jax.ShapeDtypeStruct((N_NODES, 2), jnp.float32),
        in_specs=[
            pl.BlockSpec(memory_space=pltpu.VMEM),
            pl.BlockSpec(memory_space=pltpu.VMEM),
            pl.BlockSpec(memory_space=pltpu.SMEM),
        ],
        out_specs=pl.BlockSpec(memory_space=pltpu.VMEM),
    )(h, w2, b)


def _edge_scores(pq_flat, edge_index):
    mesh = plsc.VectorSubcoreMesh(core_axis_name="c", subcore_axis_name="s")

    @functools.partial(
        pl.kernel,
        mesh=mesh,
        out_type=jax.ShapeDtypeStruct((N_EDGES,), jnp.float32),
        compiler_params=pltpu.CompilerParams(needs_layout_passes=False),
        scratch_types=[
            pltpu.VMEM((2 * N_NODES,), jnp.float32),
            pltpu.VMEM((EDGES_PER_WORKER,), jnp.int32),
            pltpu.VMEM((EDGES_PER_WORKER,), jnp.int32),
            pltpu.VMEM((EDGES_PER_WORKER,), jnp.float32),
            pltpu.SemaphoreType.DMA,
            pltpu.SemaphoreType.DMA,
            pltpu.SemaphoreType.DMA,
        ],
    )
    def sc_kernel(pq_hbm, ei_hbm, out_hbm, pq_v, src_v, dst_v, o_v, s0, s1, s2):
        wid = lax.axis_index("s") * SC_CORES + lax.axis_index("c")
        base = wid * EDGES_PER_WORKER
        sl_e = pl.ds(base, EDGES_PER_WORKER)
        c0 = pltpu.async_copy(pq_hbm, pq_v, s0)
        c1 = pltpu.async_copy(ei_hbm.at[pl.ds(base, EDGES_PER_WORKER)], src_v, s1)
        c2 = pltpu.async_copy(
            ei_hbm.at[pl.ds(N_EDGES + base, EDGES_PER_WORKER)], dst_v, s2)
        c0.wait()
        c1.wait()
        c2.wait()

        @pl.loop(0, EDGES_PER_WORKER, step=SC_LANES)
        def _(i):
            sl = pl.ds(i, SC_LANES)
            sa = src_v[sl] * 2
            da = dst_v[sl] * 2 + 1
            o_v[sl] = plsc.load_gather(pq_v, [sa]) + plsc.load_gather(pq_v, [da])

        pltpu.sync_copy(o_v, out_hbm.at[sl_e])

    return sc_kernel(pq_flat, edge_index)


def kernel(h, edge_index, W, b):
    w2 = W.reshape(2, D_FEAT).T
    pq = _node_table(h, w2, b)
    scores = _edge_scores(pq.reshape(-1), edge_index.astype(jnp.int32).reshape(-1))
    return scores.reshape(N_EDGES, 1)


# speedup vs baseline: 38.3803x; 1.0708x over previous
import functools

import jax
import jax.numpy as jnp
from jax import lax
from jax.experimental import pallas as pl
from jax.experimental.pallas import tpu as pltpu
from jax.experimental.pallas import tpu_sc as plsc

N_NODES = 10000
N_EDGES = 320000
D_FEAT = 128

SC_CORES = 2
SC_SUBCORES = 16
SC_LANES = 16
N_WORKERS = SC_CORES * SC_SUBCORES
EDGES_PER_WORKER = N_EDGES // N_WORKERS


def _node_table_body(h_ref, w2_ref, b_ref, out_ref):
    res = jnp.dot(h_ref[...], w2_ref[...], preferred_element_type=jnp.float32)
    col = lax.broadcasted_iota(jnp.int32, res.shape, 1)
    out_ref[...] = res + jnp.where(col == 0, b_ref[0], 0.0)


def _node_table(h, w2, b):
    return pl.pallas_call(
        _node_table_body,
        out_shape=jax.ShapeDtypeStruct((N_NODES, 2), jnp.float32),
        in_specs=[
            pl.BlockSpec(memory_space=pltpu.VMEM),
            pl.BlockSpec(memory_space=pltpu.VMEM),
            pl.BlockSpec(memory_space=pltpu.SMEM),
        ],
        out_specs=pl.BlockSpec(memory_space=pltpu.VMEM),
    )(h, w2, b)


def _edge_scores(pq_flat, edge_index):
    mesh = plsc.VectorSubcoreMesh(core_axis_name="c", subcore_axis_name="s")

    @functools.partial(
        pl.kernel,
        mesh=mesh,
        out_type=jax.ShapeDtypeStruct((N_EDGES,), jnp.float32),
        compiler_params=pltpu.CompilerParams(needs_layout_passes=False),
        scratch_types=[
            pltpu.VMEM((2 * N_NODES,), jnp.float32),
            pltpu.VMEM((EDGES_PER_WORKER,), jnp.int32),
            pltpu.VMEM((EDGES_PER_WORKER,), jnp.int32),
            pltpu.VMEM((EDGES_PER_WORKER,), jnp.float32),
            pltpu.VMEM_SHARED((2 * N_NODES,), jnp.float32),
            pltpu.SemaphoreType.DMA,
            pltpu.SemaphoreType.DMA,
            pltpu.SemaphoreType.DMA,
        ],
    )
    def sc_kernel(pq_hbm, ei_hbm, out_hbm, pq_v, src_v, dst_v, o_v, pq_sh,
                  s0, s1, s2):
        sid = lax.axis_index("s")
        wid = sid * SC_CORES + lax.axis_index("c")
        base = wid * EDGES_PER_WORKER
        sl_e = pl.ds(base, EDGES_PER_WORKER)
        c1 = pltpu.async_copy(ei_hbm.at[pl.ds(base, EDGES_PER_WORKER)], src_v, s1)
        c2 = pltpu.async_copy(
            ei_hbm.at[pl.ds(N_EDGES + base, EDGES_PER_WORKER)], dst_v, s2)

        @pl.when(sid == 0)
        def _():
            pltpu.sync_copy(pq_hbm, pq_sh)

        plsc.subcore_barrier()
        c0 = pltpu.async_copy(pq_sh, pq_v, s0)
        c0.wait()
        c1.wait()
        c2.wait()

        @pl.loop(0, EDGES_PER_WORKER, step=SC_LANES)
        def _(i):
            sl = pl.ds(i, SC_LANES)
            sa = src_v[sl] * 2
            da = dst_v[sl] * 2 + 1
            o_v[sl] = plsc.load_gather(pq_v, [sa]) + plsc.load_gather(pq_v, [da])

        pltpu.sync_copy(o_v, out_hbm.at[sl_e])

    return sc_kernel(pq_flat, edge_index)


def kernel(h, edge_index, W, b):
    w2 = W.reshape(2, D_FEAT).T
    pq = _node_table(h, w2, b)
    scores = _edge_scores(pq.reshape(-1), edge_index.astype(jnp.int32).reshape(-1))
    return scores.reshape(N_EDGES, 1)


# speedup vs baseline: 44.9171x; 1.1703x over previous
import functools

import jax
import jax.numpy as jnp
from jax import lax
from jax.experimental import pallas as pl
from jax.experimental.pallas import tpu as pltpu
from jax.experimental.pallas import tpu_sc as plsc

N_NODES = 10000
N_EDGES = 320000
D_FEAT = 128

SC_CORES = 2
SC_SUBCORES = 16
SC_LANES = 16
N_WORKERS = SC_CORES * SC_SUBCORES
MAIN_PER_WORKER = (N_EDGES // N_WORKERS) // 128 * 128
TAIL_BASE = N_WORKERS * MAIN_PER_WORKER
TAIL_BLOCKS = (N_EDGES - TAIL_BASE) // 128
BUF = MAIN_PER_WORKER + 128


def _node_table_body(h_ref, wt_ref, b_ref, out_ref):
    res = lax.dot_general(
        wt_ref[...], h_ref[...],
        dimension_numbers=(((1,), (1,)), ((), ())),
        preferred_element_type=jnp.float32,
    )
    row = lax.broadcasted_iota(jnp.int32, res.shape, 0)
    out_ref[...] = res + jnp.where(row == 0, b_ref[0], 0.0)


def _node_table(h, wt, b):
    return pl.pallas_call(
        _node_table_body,
        out_shape=jax.ShapeDtypeStruct((2, N_NODES), jnp.float32),
        in_specs=[
            pl.BlockSpec(memory_space=pltpu.VMEM),
            pl.BlockSpec(memory_space=pltpu.VMEM),
            pl.BlockSpec(memory_space=pltpu.SMEM),
        ],
        out_specs=pl.BlockSpec(memory_space=pltpu.VMEM),
    )(h, wt, b)


def _edge_scores(pq, edge_index):
    mesh = plsc.VectorSubcoreMesh(core_axis_name="c", subcore_axis_name="s")

    @functools.partial(
        pl.kernel,
        mesh=mesh,
        out_type=jax.ShapeDtypeStruct((N_EDGES,), jnp.float32),
        compiler_params=pltpu.CompilerParams(needs_layout_passes=False),
        scratch_types=[
            pltpu.VMEM((2, N_NODES), jnp.float32),
            pltpu.VMEM((2, BUF), jnp.int32),
            pltpu.VMEM((BUF,), jnp.float32),
            pltpu.VMEM_SHARED((2, N_NODES), jnp.float32),
            pltpu.SemaphoreType.DMA,
            pltpu.SemaphoreType.DMA,
            pltpu.SemaphoreType.DMA,
        ],
    )
    def sc_kernel(pq_hbm, ei_hbm, out_hbm, pq_v, sd_v, o_v, pq_sh, s0, s1, s2):
        sid = lax.axis_index("s")
        wid = sid * SC_CORES + lax.axis_index("c")
        base = wid * MAIN_PER_WORKER
        tail = TAIL_BASE + wid * 128
        has_tail = wid < TAIL_BLOCKS
        c1 = pltpu.async_copy(
            ei_hbm.at[:, pl.ds(base, MAIN_PER_WORKER)],
            sd_v.at[:, pl.ds(0, MAIN_PER_WORKER)], s1)

        @pl.when(has_tail)
        def _():
            pltpu.async_copy(
                ei_hbm.at[:, pl.ds(tail, 128)],
                sd_v.at[:, pl.ds(MAIN_PER_WORKER, 128)], s2).wait()

        @pl.when(sid == 0)
        def _():
            pltpu.sync_copy(pq_hbm, pq_sh)

        plsc.subcore_barrier()
        c0 = pltpu.async_copy(pq_sh, pq_v, s0)
        c0.wait()
        c1.wait()

        zero = jnp.zeros((SC_LANES,), jnp.int32)
        one = jnp.ones((SC_LANES,), jnp.int32)

        def score_block(i):
            sl = pl.ds(i, SC_LANES)
            pv = plsc.load_gather(pq_v, [zero, sd_v[0, sl]])
            qv = plsc.load_gather(pq_v, [one, sd_v[1, sl]])
            o_v[sl] = pv + qv

        pl.loop(0, MAIN_PER_WORKER, step=SC_LANES)(score_block)
        pltpu.sync_copy(
            o_v.at[pl.ds(0, MAIN_PER_WORKER)],
            out_hbm.at[pl.ds(base, MAIN_PER_WORKER)])

        @pl.when(has_tail)
        def _():
            pl.loop(MAIN_PER_WORKER, BUF, step=SC_LANES)(score_block)
            pltpu.sync_copy(
                o_v.at[pl.ds(MAIN_PER_WORKER, 128)],
                out_hbm.at[pl.ds(tail, 128)])

    return sc_kernel(pq, edge_index).reshape(N_EDGES, 1)


def kernel(h, edge_index, W, b):
    wt = W.reshape(2, D_FEAT)
    pq = _node_table(h, wt, b)
    return _edge_scores(pq, edge_index.astype(jnp.int32))


# speedup vs baseline: 45.0227x; 1.0024x over previous
import functools

import jax
import jax.numpy as jnp
from jax import lax
from jax.experimental import pallas as pl
from jax.experimental.pallas import tpu as pltpu
from jax.experimental.pallas import tpu_sc as plsc

N_NODES = 10000
N_EDGES = 320000
D_FEAT = 128

SC_CORES = 2
SC_SUBCORES = 16
SC_LANES = 16
N_WORKERS = SC_CORES * SC_SUBCORES
MAIN_PER_WORKER = (N_EDGES // N_WORKERS) // 128 * 128
TAIL_BASE = N_WORKERS * MAIN_PER_WORKER
TAIL_BLOCKS = (N_EDGES - TAIL_BASE) // 128
BUF = MAIN_PER_WORKER + 128
N_PAD = (N_NODES + 127) // 128 * 128


def _node_table_body(h_ref, wt_ref, b_ref, out_ref):
    res = lax.dot_general(
        wt_ref[...], h_ref[...],
        dimension_numbers=(((1,), (1,)), ((), ())),
        preferred_element_type=jnp.float32,
    )
    row = lax.broadcasted_iota(jnp.int32, res.shape, 0)
    out_ref[:, pl.ds(0, N_NODES)] = res + jnp.where(row == 0, b_ref[0], 0.0)


def _node_table(h, wt, b):
    return pl.pallas_call(
        _node_table_body,
        out_shape=jax.ShapeDtypeStruct((2, N_PAD), jnp.float32),
        in_specs=[
            pl.BlockSpec(memory_space=pltpu.VMEM),
            pl.BlockSpec(memory_space=pltpu.VMEM),
            pl.BlockSpec(memory_space=pltpu.SMEM),
        ],
        out_specs=pl.BlockSpec(memory_space=pltpu.VMEM),
    )(h, wt, b)


def _edge_scores(pq, edge_index):
    mesh = plsc.VectorSubcoreMesh(core_axis_name="c", subcore_axis_name="s")

    @functools.partial(
        pl.kernel,
        mesh=mesh,
        out_type=jax.ShapeDtypeStruct((N_EDGES,), jnp.float32),
        compiler_params=pltpu.CompilerParams(needs_layout_passes=False),
        scratch_types=[
            pltpu.VMEM((2, N_PAD), jnp.float32),
            pltpu.VMEM((2, BUF), jnp.int32),
            pltpu.VMEM((BUF,), jnp.float32),
            pltpu.VMEM_SHARED((2, N_PAD), jnp.float32),
            pltpu.SemaphoreType.DMA,
            pltpu.SemaphoreType.DMA,
            pltpu.SemaphoreType.DMA,
        ],
    )
    def sc_kernel(pq_hbm, ei_hbm, out_hbm, pq_v, sd_v, o_v, pq_sh, s0, s1, s2):
        sid = lax.axis_index("s")
        wid = sid * SC_CORES + lax.axis_index("c")
        base = wid * MAIN_PER_WORKER
        tail = TAIL_BASE + wid * 128
        has_tail = wid < TAIL_BLOCKS
        c1 = pltpu.async_copy(
            ei_hbm.at[:, pl.ds(base, MAIN_PER_WORKER)],
            sd_v.at[:, pl.ds(0, MAIN_PER_WORKER)], s1)

        @pl.when(has_tail)
        def _():
            pltpu.async_copy(
                ei_hbm.at[:, pl.ds(tail, 128)],
                sd_v.at[:, pl.ds(MAIN_PER_WORKER, 128)], s2).wait()

        @pl.when(sid == 0)
        def _():
            pltpu.sync_copy(pq_hbm, pq_sh)

        plsc.subcore_barrier()
        c0 = pltpu.async_copy(pq_sh, pq_v, s0)
        c0.wait()
        c1.wait()

        zero = jnp.zeros((SC_LANES,), jnp.int32)
        one = jnp.ones((SC_LANES,), jnp.int32)

        def score_block(i):
            sl = pl.ds(i, SC_LANES)
            pv = plsc.load_gather(pq_v, [zero, sd_v[0, sl]])
            qv = plsc.load_gather(pq_v, [one, sd_v[1, sl]])
            o_v[sl] = pv + qv

        pl.loop(0, MAIN_PER_WORKER, step=SC_LANES)(score_block)
        pltpu.sync_copy(
            o_v.at[pl.ds(0, MAIN_PER_WORKER)],
            out_hbm.at[pl.ds(base, MAIN_PER_WORKER)])

        @pl.when(has_tail)
        def _():
            pl.loop(MAIN_PER_WORKER, BUF, step=SC_LANES)(score_block)
            pltpu.sync_copy(
                o_v.at[pl.ds(MAIN_PER_WORKER, 128)],
                out_hbm.at[pl.ds(tail, 128)])

    return sc_kernel(pq, edge_index).reshape(N_EDGES, 1)


def kernel(h, edge_index, W, b):
    wt = W.reshape(2, D_FEAT)
    pq = _node_table(h, wt, b)
    return _edge_scores(pq, edge_index.astype(jnp.int32))


# speedup vs baseline: 47.1826x; 1.0480x over previous
import functools

import jax
import jax.numpy as jnp
from jax import lax
from jax.experimental import pallas as pl
from jax.experimental.pallas import tpu as pltpu
from jax.experimental.pallas import tpu_sc as plsc

N_NODES = 10000
N_EDGES = 320000
D_FEAT = 128

SC_CORES = 2
SC_SUBCORES = 16
SC_LANES = 16
N_WORKERS = SC_CORES * SC_SUBCORES
MAIN_PER_WORKER = (N_EDGES // N_WORKERS) // 128 * 128
TAIL_BASE = N_WORKERS * MAIN_PER_WORKER
TAIL_BLOCKS = (N_EDGES - TAIL_BASE) // 128
BUF = MAIN_PER_WORKER + 128
N_PAD = (N_NODES + 127) // 128 * 128


def _node_table_body(h_ref, wt_ref, b_ref, out_ref):
    res = lax.dot_general(
        wt_ref[...], h_ref[...],
        dimension_numbers=(((1,), (1,)), ((), ())),
        preferred_element_type=jnp.float32,
    )
    row = lax.broadcasted_iota(jnp.int32, res.shape, 0)
    out_ref[:, pl.ds(0, N_NODES)] = res + jnp.where(row == 0, b_ref[0], 0.0)


def _node_table(h, wt, b):
    return pl.pallas_call(
        _node_table_body,
        out_shape=jax.ShapeDtypeStruct((2, N_PAD), jnp.float32),
        in_specs=[
            pl.BlockSpec(memory_space=pltpu.VMEM),
            pl.BlockSpec(memory_space=pltpu.VMEM),
            pl.BlockSpec(memory_space=pltpu.SMEM),
        ],
        out_specs=pl.BlockSpec(memory_space=pltpu.VMEM),
    )(h, wt, b)


def _edge_scores(pq, edge_index):
    mesh = plsc.VectorSubcoreMesh(core_axis_name="c", subcore_axis_name="s")

    @functools.partial(
        pl.kernel,
        mesh=mesh,
        out_type=jax.ShapeDtypeStruct((N_EDGES,), jnp.float32),
        compiler_params=pltpu.CompilerParams(needs_layout_passes=False),
        scratch_types=[
            pltpu.VMEM((N_PAD,), jnp.float32),
            pltpu.VMEM((N_PAD,), jnp.float32),
            pltpu.VMEM((2, BUF), jnp.int32),
            pltpu.VMEM((BUF,), jnp.float32),
            pltpu.VMEM_SHARED((2, N_PAD), jnp.float32),
            pltpu.SemaphoreType.DMA,
            pltpu.SemaphoreType.DMA,
            pltpu.SemaphoreType.DMA,
            pltpu.SemaphoreType.DMA,
        ],
    )
    def sc_kernel(pq_hbm, ei_hbm, out_hbm, p_v, q_v, sd_v, o_v, pq_sh,
                  s0, s1, s2, s3):
        sid = lax.axis_index("s")
        wid = sid * SC_CORES + lax.axis_index("c")
        base = wid * MAIN_PER_WORKER
        tail = TAIL_BASE + wid * 128
        has_tail = wid < TAIL_BLOCKS
        c1 = pltpu.async_copy(
            ei_hbm.at[:, pl.ds(base, MAIN_PER_WORKER)],
            sd_v.at[:, pl.ds(0, MAIN_PER_WORKER)], s1)

        @pl.when(has_tail)
        def _():
            pltpu.async_copy(
                ei_hbm.at[:, pl.ds(tail, 128)],
                sd_v.at[:, pl.ds(MAIN_PER_WORKER, 128)], s2).wait()

        half = (N_PAD // 128 // 2) * 128

        @pl.when(sid == 0)
        def _():
            pltpu.sync_copy(pq_hbm.at[:, pl.ds(0, half)],
                            pq_sh.at[:, pl.ds(0, half)])

        @pl.when(sid == 1)
        def _():
            pltpu.sync_copy(pq_hbm.at[:, pl.ds(half, N_PAD - half)],
                            pq_sh.at[:, pl.ds(half, N_PAD - half)])

        plsc.subcore_barrier()
        c0 = pltpu.async_copy(pq_sh.at[0], p_v, s0)
        c3 = pltpu.async_copy(pq_sh.at[1], q_v, s3)
        c0.wait()
        c3.wait()
        c1.wait()

        def score_block(i):
            sl = pl.ds(i, SC_LANES)
            pv = plsc.load_gather(p_v, [sd_v[0, sl]])
            qv = plsc.load_gather(q_v, [sd_v[1, sl]])
            o_v[sl] = pv + qv

        pl.loop(0, MAIN_PER_WORKER, step=SC_LANES)(score_block)
        pltpu.sync_copy(
            o_v.at[pl.ds(0, MAIN_PER_WORKER)],
            out_hbm.at[pl.ds(base, MAIN_PER_WORKER)])

        @pl.when(has_tail)
        def _():
            pl.loop(MAIN_PER_WORKER, BUF, step=SC_LANES)(score_block)
            pltpu.sync_copy(
                o_v.at[pl.ds(MAIN_PER_WORKER, 128)],
                out_hbm.at[pl.ds(tail, 128)])

    return sc_kernel(pq, edge_index).reshape(N_EDGES, 1)


def kernel(h, edge_index, W, b):
    wt = W.reshape(2, D_FEAT)
    pq = _node_table(h, wt, b)
    return _edge_scores(pq, edge_index.astype(jnp.int32))


# speedup vs baseline: 48.8203x; 1.0347x over previous
import functools

import jax
import jax.numpy as jnp
from jax import lax
from jax.experimental import pallas as pl
from jax.experimental.pallas import tpu as pltpu
from jax.experimental.pallas import tpu_sc as plsc

N_NODES = 10000
N_EDGES = 320000
D_FEAT = 128

SC_CORES = 2
SC_SUBCORES = 16
SC_LANES = 16
N_WORKERS = SC_CORES * SC_SUBCORES
MAIN_PER_WORKER = (N_EDGES // N_WORKERS) // 128 * 128
TAIL_BASE = N_WORKERS * MAIN_PER_WORKER
TAIL_BLOCKS = (N_EDGES - TAIL_BASE) // 128
BUF = MAIN_PER_WORKER + 128
N_PAD = 10240


def _node_table_body(h_ref, wt_ref, b_ref, out_ref):
    res = lax.dot_general(
        wt_ref[...], h_ref[...],
        dimension_numbers=(((1,), (1,)), ((), ())),
        preferred_element_type=jnp.float32,
    )
    row = lax.broadcasted_iota(jnp.int32, res.shape, 0)
    out_ref[...] = res + jnp.where(row == 0, b_ref[0], 0.0)


def _node_table(h, wt, b):
    blk = 1024
    return pl.pallas_call(
        _node_table_body,
        grid=(N_PAD // blk,),
        out_shape=jax.ShapeDtypeStruct((2, N_PAD), jnp.float32),
        in_specs=[
            pl.BlockSpec((blk, D_FEAT), lambda i: (i, 0)),
            pl.BlockSpec((2, D_FEAT), lambda i: (0, 0)),
            pl.BlockSpec(memory_space=pltpu.SMEM),
        ],
        out_specs=pl.BlockSpec((2, blk), lambda i: (0, i)),
    )(h, wt, b)


def _edge_scores(pq, edge_index):
    mesh = plsc.VectorSubcoreMesh(core_axis_name="c", subcore_axis_name="s")

    @functools.partial(
        pl.kernel,
        mesh=mesh,
        out_type=jax.ShapeDtypeStruct((N_EDGES,), jnp.float32),
        compiler_params=pltpu.CompilerParams(needs_layout_passes=False),
        scratch_types=[
            pltpu.VMEM((N_PAD,), jnp.float32),
            pltpu.VMEM((N_PAD,), jnp.float32),
            pltpu.VMEM((2, BUF), jnp.int32),
            pltpu.VMEM((BUF,), jnp.float32),
            pltpu.VMEM_SHARED((2, N_PAD), jnp.float32),
            pltpu.SemaphoreType.DMA,
            pltpu.SemaphoreType.DMA,
            pltpu.SemaphoreType.DMA,
            pltpu.SemaphoreType.DMA,
        ],
    )
    def sc_kernel(pq_hbm, ei_hbm, out_hbm, p_v, q_v, sd_v, o_v, pq_sh,
                  s0, s1, s2, s3):
        sid = lax.axis_index("s")
        wid = sid * SC_CORES + lax.axis_index("c")
        base = wid * MAIN_PER_WORKER
        tail = TAIL_BASE + wid * 128
        has_tail = wid < TAIL_BLOCKS
        c1 = pltpu.async_copy(
            ei_hbm.at[:, pl.ds(base, MAIN_PER_WORKER)],
            sd_v.at[:, pl.ds(0, MAIN_PER_WORKER)], s1)

        @pl.when(has_tail)
        def _():
            pltpu.async_copy(
                ei_hbm.at[:, pl.ds(tail, 128)],
                sd_v.at[:, pl.ds(MAIN_PER_WORKER, 128)], s2).wait()

        half = (N_PAD // 128 // 2) * 128

        @pl.when(sid == 0)
        def _():
            pltpu.sync_copy(pq_hbm.at[:, pl.ds(0, half)],
                            pq_sh.at[:, pl.ds(0, half)])

        @pl.when(sid == 1)
        def _():
            pltpu.sync_copy(pq_hbm.at[:, pl.ds(half, N_PAD - half)],
                            pq_sh.at[:, pl.ds(half, N_PAD - half)])

        plsc.subcore_barrier()
        c0 = pltpu.async_copy(pq_sh.at[0], p_v, s0)
        c3 = pltpu.async_copy(pq_sh.at[1], q_v, s3)
        c0.wait()
        c3.wait()
        c1.wait()

        def score_block(i):
            sl = pl.ds(i, SC_LANES)
            pv = plsc.load_gather(p_v, [sd_v[0, sl]])
            qv = plsc.load_gather(q_v, [sd_v[1, sl]])
            o_v[sl] = pv + qv

        plsc.parallel_loop(0, MAIN_PER_WORKER, step=SC_LANES, unroll=4)(score_block)
        pltpu.sync_copy(
            o_v.at[pl.ds(0, MAIN_PER_WORKER)],
            out_hbm.at[pl.ds(base, MAIN_PER_WORKER)])

        @pl.when(has_tail)
        def _():
            plsc.parallel_loop(MAIN_PER_WORKER, BUF, step=SC_LANES, unroll=4)(score_block)
            pltpu.sync_copy(
                o_v.at[pl.ds(MAIN_PER_WORKER, 128)],
                out_hbm.at[pl.ds(tail, 128)])

    return sc_kernel(pq, edge_index).reshape(N_EDGES, 1)


def kernel(h, edge_index, W, b):
    wt = W.reshape(2, D_FEAT)
    pq = _node_table(h, wt, b)
    return _edge_scores(pq, edge_index.astype(jnp.int32))


# speedup vs baseline: 53.1705x; 1.0891x over previous
import functools

import jax
import jax.numpy as jnp
from jax import lax
from jax.experimental import pallas as pl
from jax.experimental.pallas import tpu as pltpu
from jax.experimental.pallas import tpu_sc as plsc

N_NODES = 10000
N_EDGES = 320000
D_FEAT = 128

SC_CORES = 2
SC_SUBCORES = 16
SC_LANES = 16
N_WORKERS = SC_CORES * SC_SUBCORES
MAIN_PER_WORKER = (N_EDGES // N_WORKERS) // 128 * 128
TAIL_BASE = N_WORKERS * MAIN_PER_WORKER
TAIL_BLOCKS = (N_EDGES - TAIL_BASE) // 128
BUF = MAIN_PER_WORKER + 128
N_PAD = 10240


def _node_table_body(h_ref, wt_ref, b_ref, out_ref):
    res = lax.dot_general(
        wt_ref[...], h_ref[...],
        dimension_numbers=(((1,), (1,)), ((), ())),
        preferred_element_type=jnp.float32,
    )
    row = lax.broadcasted_iota(jnp.int32, res.shape, 0)
    out_ref[...] = res + jnp.where(row == 0, b_ref[0], 0.0)


def _node_table(h, wt, b):
    blk = 2560
    return pl.pallas_call(
        _node_table_body,
        grid=(N_PAD // blk,),
        out_shape=jax.ShapeDtypeStruct((2, N_PAD), jnp.float32),
        in_specs=[
            pl.BlockSpec((blk, D_FEAT), lambda i: (i, 0)),
            pl.BlockSpec((2, D_FEAT), lambda i: (0, 0)),
            pl.BlockSpec(memory_space=pltpu.SMEM),
        ],
        out_specs=pl.BlockSpec((2, blk), lambda i: (0, i)),
    )(h, wt, b)


def _edge_scores(pq, edge_index):
    mesh = plsc.VectorSubcoreMesh(core_axis_name="c", subcore_axis_name="s")

    @functools.partial(
        pl.kernel,
        mesh=mesh,
        out_type=jax.ShapeDtypeStruct((N_EDGES,), jnp.float32),
        compiler_params=pltpu.CompilerParams(needs_layout_passes=False),
        scratch_types=[
            pltpu.VMEM((N_PAD,), jnp.float32),
            pltpu.VMEM((N_PAD,), jnp.float32),
            pltpu.VMEM((2, BUF), jnp.int32),
            pltpu.VMEM((BUF,), jnp.float32),
            pltpu.VMEM_SHARED((2, N_PAD), jnp.float32),
            pltpu.SemaphoreType.DMA,
            pltpu.SemaphoreType.DMA,
            pltpu.SemaphoreType.DMA,
            pltpu.SemaphoreType.DMA,
        ],
    )
    def sc_kernel(pq_hbm, ei_hbm, out_hbm, p_v, q_v, sd_v, o_v, pq_sh,
                  s0, s1, s2, s3):
        sid = lax.axis_index("s")
        wid = sid * SC_CORES + lax.axis_index("c")
        base = wid * MAIN_PER_WORKER
        tail = TAIL_BASE + wid * 128
        has_tail = wid < TAIL_BLOCKS
        c1 = pltpu.async_copy(
            ei_hbm.at[:, pl.ds(base, MAIN_PER_WORKER)],
            sd_v.at[:, pl.ds(0, MAIN_PER_WORKER)], s1)

        @pl.when(has_tail)
        def _():
            pltpu.async_copy(
                ei_hbm.at[:, pl.ds(tail, 128)],
                sd_v.at[:, pl.ds(MAIN_PER_WORKER, 128)], s2).wait()

        half = (N_PAD // 128 // 2) * 128

        @pl.when(sid == 0)
        def _():
            pltpu.sync_copy(pq_hbm.at[:, pl.ds(0, half)],
                            pq_sh.at[:, pl.ds(0, half)])

        @pl.when(sid == 1)
        def _():
            pltpu.sync_copy(pq_hbm.at[:, pl.ds(half, N_PAD - half)],
                            pq_sh.at[:, pl.ds(half, N_PAD - half)])

        plsc.subcore_barrier()
        c0 = pltpu.async_copy(pq_sh.at[0], p_v, s0)
        c3 = pltpu.async_copy(pq_sh.at[1], q_v, s3)
        c0.wait()
        c3.wait()
        c1.wait()

        def score_block(i):
            sl = pl.ds(i, SC_LANES)
            pv = plsc.load_gather(p_v, [sd_v[0, sl]])
            qv = plsc.load_gather(q_v, [sd_v[1, sl]])
            o_v[sl] = pv + qv

        plsc.parallel_loop(0, MAIN_PER_WORKER, step=SC_LANES, unroll=4)(score_block)
        pltpu.sync_copy(
            o_v.at[pl.ds(0, MAIN_PER_WORKER)],
            out_hbm.at[pl.ds(base, MAIN_PER_WORKER)])

        @pl.when(has_tail)
        def _():
            plsc.parallel_loop(MAIN_PER_WORKER, BUF, step=SC_LANES, unroll=4)(score_block)
            pltpu.sync_copy(
                o_v.at[pl.ds(MAIN_PER_WORKER, 128)],
                out_hbm.at[pl.ds(tail, 128)])

    return sc_kernel(pq, edge_index).reshape(N_EDGES, 1)


def kernel(h, edge_index, W, b):
    wt = W.reshape(2, D_FEAT)
    pq = _node_table(h, wt, b)
    return _edge_scores(pq, edge_index.astype(jnp.int32))


# speedup vs baseline: 54.7676x; 1.0300x over previous
import functools

import jax
import jax.numpy as jnp
from jax import lax
from jax.experimental import pallas as pl
from jax.experimental.pallas import tpu as pltpu
from jax.experimental.pallas import tpu_sc as plsc

N_NODES = 10000
N_EDGES = 320000
D_FEAT = 128

SC_CORES = 2
SC_SUBCORES = 16
SC_LANES = 16
N_WORKERS = SC_CORES * SC_SUBCORES
MAIN_PER_WORKER = (N_EDGES // N_WORKERS) // 128 * 128
TAIL_BASE = N_WORKERS * MAIN_PER_WORKER
TAIL_BLOCKS = (N_EDGES - TAIL_BASE) // 128
BUF = MAIN_PER_WORKER + 128
N_PAD = 10240


def _node_table_body(h_ref, wt_ref, b_ref, out_ref):
    res = lax.dot_general(
        wt_ref[...], h_ref[...],
        dimension_numbers=(((1,), (1,)), ((), ())),
        preferred_element_type=jnp.float32,
    )
    row = lax.broadcasted_iota(jnp.int32, res.shape, 0)
    out_ref[...] = res + jnp.where(row == 0, b_ref[0], 0.0)


def _node_table(h, wt, b):
    blk = 5120
    return pl.pallas_call(
        _node_table_body,
        grid=(N_PAD // blk,),
        out_shape=jax.ShapeDtypeStruct((2, N_PAD), jnp.float32),
        in_specs=[
            pl.BlockSpec((blk, D_FEAT), lambda i: (i, 0)),
            pl.BlockSpec((2, D_FEAT), lambda i: (0, 0)),
            pl.BlockSpec(memory_space=pltpu.SMEM),
        ],
        out_specs=pl.BlockSpec((2, blk), lambda i: (0, i)),
    )(h, wt, b)


def _edge_scores(pq, edge_index):
    mesh = plsc.VectorSubcoreMesh(core_axis_name="c", subcore_axis_name="s")

    @functools.partial(
        pl.kernel,
        mesh=mesh,
        out_type=jax.ShapeDtypeStruct((N_EDGES,), jnp.float32),
        compiler_params=pltpu.CompilerParams(needs_layout_passes=False),
        scratch_types=[
            pltpu.VMEM((N_PAD,), jnp.float32),
            pltpu.VMEM((N_PAD,), jnp.float32),
            pltpu.VMEM((2, BUF), jnp.int32),
            pltpu.VMEM((BUF,), jnp.float32),
            pltpu.VMEM_SHARED((2, N_PAD), jnp.float32),
            pltpu.SemaphoreType.DMA,
            pltpu.SemaphoreType.DMA,
            pltpu.SemaphoreType.DMA,
            pltpu.SemaphoreType.DMA,
        ],
    )
    def sc_kernel(pq_hbm, ei_hbm, out_hbm, p_v, q_v, sd_v, o_v, pq_sh,
                  s0, s1, s2, s3):
        sid = lax.axis_index("s")
        wid = sid * SC_CORES + lax.axis_index("c")
        base = wid * MAIN_PER_WORKER
        tail = TAIL_BASE + wid * 128
        has_tail = wid < TAIL_BLOCKS
        c1 = pltpu.async_copy(
            ei_hbm.at[:, pl.ds(base, MAIN_PER_WORKER)],
            sd_v.at[:, pl.ds(0, MAIN_PER_WORKER)], s1)

        @pl.when(has_tail)
        def _():
            pltpu.async_copy(
                ei_hbm.at[:, pl.ds(tail, 128)],
                sd_v.at[:, pl.ds(MAIN_PER_WORKER, 128)], s2).wait()

        half = (N_PAD // 128 // 2) * 128

        @pl.when(sid == 0)
        def _():
            pltpu.sync_copy(pq_hbm.at[:, pl.ds(0, half)],
                            pq_sh.at[:, pl.ds(0, half)])

        @pl.when(sid == 1)
        def _():
            pltpu.sync_copy(pq_hbm.at[:, pl.ds(half, N_PAD - half)],
                            pq_sh.at[:, pl.ds(half, N_PAD - half)])

        plsc.subcore_barrier()
        c0 = pltpu.async_copy(pq_sh.at[0], p_v, s0)
        c3 = pltpu.async_copy(pq_sh.at[1], q_v, s3)
        c0.wait()
        c3.wait()
        c1.wait()

        def score_block(i):
            sl = pl.ds(i, SC_LANES)
            pv = plsc.load_gather(p_v, [sd_v[0, sl]])
            qv = plsc.load_gather(q_v, [sd_v[1, sl]])
            o_v[sl] = pv + qv

        plsc.parallel_loop(0, MAIN_PER_WORKER, step=SC_LANES, unroll=4)(score_block)
        pltpu.sync_copy(
            o_v.at[pl.ds(0, MAIN_PER_WORKER)],
            out_hbm.at[pl.ds(base, MAIN_PER_WORKER)])

        @pl.when(has_tail)
        def _():
            plsc.parallel_loop(MAIN_PER_WORKER, BUF, step=SC_LANES, unroll=4)(score_block)
            pltpu.sync_copy(
                o_v.at[pl.ds(MAIN_PER_WORKER, 128)],
                out_hbm.at[pl.ds(tail, 128)])

    return sc_kernel(pq, edge_index).reshape(N_EDGES, 1)


def kernel(h, edge_index, W, b):
    wt = W.reshape(2, D_FEAT)
    pq = _node_table(h, wt, b)
    return _edge_scores(pq, edge_index.astype(jnp.int32))
